# baseline (speedup 1.0000x reference)
import functools

import jax
import jax.numpy as jnp
from jax import lax
from jax.experimental import pallas as pl
from jax.experimental.pallas import tpu as pltpu
from jax.experimental.pallas import tpu_sc as plsc

B, H, W = 4, 512, 512
HW = H * W
KH = KW = 32
K = KH * KW
YX_SCALE = KH / (0.4 * H)
LAB_SCALE = 0.26

NC, NS = 2, 16
NW = NC * NS
PW = (B * HW) // NW
WPB = NW // B
CH = 4096
NCHUNK = PW // CH
NV = CH // 16

_OFFS = ((-1, -1), (-1, 0), (-1, 1), (0, -1), (0, 0), (0, 1), (1, -1), (1, 0), (1, 1))


def _mesh():
    return plsc.VectorSubcoreMesh(core_axis_name="c", subcore_axis_name="s")


_CPARAMS = pltpu.CompilerParams(
    use_tc_tiling_on_sc=False, needs_layout_passes=False)


def _worker_id():
    return lax.axis_index("s") * NC + lax.axis_index("c")


def _tree_sum(xs):
    while len(xs) > 1:
        xs = [xs[i] + xs[i + 1] for i in range(0, len(xs) - 1, 2)] \
            + ([xs[-1]] if len(xs) % 2 else [])
    return xs[0]


def _tree_min(xs):
    while len(xs) > 1:
        xs = [jnp.minimum(xs[i], xs[i + 1]) for i in range(0, len(xs) - 1, 2)] \
            + ([xs[-1]] if len(xs) % 2 else [])
    return xs[0]


def _nbr_indices(s):
    sh = lax.shift_right_logical(s, 5)
    sw = lax.bitwise_and(s, KW - 1)
    out = []
    for dh, dw in _OFFS:
        nh = sh
        if dh < 0:
            nh = jnp.maximum(sh - 1, 0)
        elif dh > 0:
            nh = jnp.minimum(sh + 1, KH - 1)
        nw = sw
        if dw < 0:
            nw = jnp.maximum(sw - 1, 0)
        elif dw > 0:
            nw = jnp.minimum(sw + 1, KW - 1)
        out.append(lax.bitwise_or(lax.shift_left(nh, 5), nw))
    return out


def _pixel_feats(pix_base, i, bufs):
    _, l0b, l1b, l2b = bufs
    lane = lax.iota(jnp.int32, 16)
    pix = (pix_base + i * 16) + lane
    y = lax.shift_right_logical(pix, 9).astype(jnp.float32) * YX_SCALE
    x = lax.bitwise_and(pix, W - 1).astype(jnp.float32) * YX_SCALE
    sl = pl.ds(i * 16, 16)
    return y, x, l0b[sl], l1b[sl], l2b[sl]


def _zero_acc(acc):
    zero = jnp.zeros((16,), jnp.float32)
    for c in range(6):
        def zbody(i, _, c=c):
            acc[c, pl.ds(i * 16, 16)] = zero
            return 0
        lax.fori_loop(0, K // 16, zbody, 0)


def _reduce_partials(part_in, b, part8, sums):
    pltpu.sync_copy(part_in.at[pl.ds(b * WPB, WPB)], part8)
    for c in range(6):
        def rbody(i, _, c=c):
            sl = pl.ds(i * 16, 16)
            sums[c, sl] = _tree_sum([part8[t, c, sl] for t in range(WPB)])
            return 0
        lax.fori_loop(0, K // 16, rbody, 0)


def _build_table(sums, tab, mode):
    def tbody(i, _):
        sl = pl.ds(i * 16, 16)
        den = sums[5, sl]
        if mode == "init":
            den = jnp.maximum(den, 1e-12)
        else:
            den = den + 1e-10
        r = 1.0 / den
        h = None
        for c in range(5):
            t = sums[c, sl] * r
            tab[c, sl] = -2.0 * t
            t2 = t * t
            h = t2 if h is None else h + t2
        tab[5, sl] = h
        return 0
    lax.fori_loop(0, K // 16, tbody, 0)


def _gather_dists(tab, feats, nidx):
    dists = []
    for n in nidx:
        terms = [plsc.load_gather(tab, [jnp.full((16,), 5, jnp.int32), n])]
        for c, f in enumerate(feats):
            g = plsc.load_gather(tab, [jnp.full((16,), c, jnp.int32), n])
            terms.append(f * g)
        dists.append(_tree_sum(terms))
    return dists


def _softmax9(dists):
    m = _tree_min(dists)
    es = [jnp.exp(m - d) for d in dists]
    r = 1.0 / _tree_sum(es)
    return [e * r for e in es]


_IN_CHUNKS = [
    pltpu.VMEM((CH,), jnp.int32),
    pltpu.VMEM((CH,), jnp.float32),
    pltpu.VMEM((CH,), jnp.float32),
    pltpu.VMEM((CH,), jnp.float32),
]


def _stream_chunks(srcs, g0, scratches, per_chunk):
    bufs = tuple(scratches[0:4])

    def chunk_body(cki, _):
        off = cki * CH
        for src, dst in zip(srcs, bufs):
            pltpu.sync_copy(src.at[pl.ds(g0 + off, CH)], dst)
        per_chunk(cki, off, bufs)
        return 0

    lax.fori_loop(0, NCHUNK, chunk_body, 0)


@functools.partial(
    pl.kernel,
    out_type=jax.ShapeDtypeStruct((NW, 6, K), jnp.float32),
    mesh=_mesh(),
    compiler_params=_CPARAMS,
    scratch_types=[pltpu.VMEM((6, K), jnp.float32)] + _IN_CHUNKS,
)
def _init_gather(spx_hbm, lab0, lab1, lab2, part_out, acc, *scratches):
    wid = _worker_id()
    g0 = wid * PW
    pix_base0 = (wid % WPB) * PW
    _zero_acc(acc)
    one = jnp.ones((16,), jnp.float32)
    c5 = jnp.full((16,), 5, jnp.int32)

    def per_chunk(cki, off, bufs):
        def vbody(i, _):
            s = bufs[0][pl.ds(i * 16, 16)]
            feats = _pixel_feats(pix_base0 + off, i, bufs)
            for c, f in enumerate(feats):
                plsc.addupdate_scatter(acc, [jnp.full((16,), c, jnp.int32), s], f)
            plsc.addupdate_scatter(acc, [c5, s], one)
            return 0
        lax.fori_loop(0, NV, vbody, 0)

    _stream_chunks((spx_hbm, lab0, lab1, lab2), g0, scratches, per_chunk)
    pltpu.sync_copy(acc, part_out.at[wid])


def _make_step(mode):
    @functools.partial(
        pl.kernel,
        out_type=jax.ShapeDtypeStruct((NW, 6, K), jnp.float32),
        mesh=_mesh(),
        compiler_params=_CPARAMS,
        scratch_types=[
            pltpu.VMEM((WPB, 6, K), jnp.float32),
            pltpu.VMEM((6, K), jnp.float32),
            pltpu.VMEM((6, K), jnp.float32),
            pltpu.VMEM((6, K), jnp.float32),
        ] + _IN_CHUNKS,
    )
    def step(part_in, spx_hbm, lab0, lab1, lab2, part_out,
             part8, sums, tab, acc, *scratches):
        wid = _worker_id()
        b = wid // WPB
        g0 = wid * PW
        pix_base0 = (wid % WPB) * PW
        _reduce_partials(part_in, b, part8, sums)
        _build_table(sums, tab, mode)
        _zero_acc(acc)
        c5 = jnp.full((16,), 5, jnp.int32)

        def per_chunk(cki, off, bufs):
            def vbody(i):
                s = bufs[0][pl.ds(i * 16, 16)]
                feats = _pixel_feats(pix_base0 + off, i, bufs)
                nidx = _nbr_indices(s)
                dists = _gather_dists(tab, feats, nidx)
                ws = _softmax9(dists)
                for j, n in enumerate(nidx):
                    w = ws[j]
                    plsc.addupdate_scatter(acc, [c5, n], w)
                    for c, f in enumerate(feats):
                        plsc.addupdate_scatter(
                            acc, [jnp.full((16,), c, jnp.int32), n], w * f)
            plsc.parallel_loop(0, NV, unroll=2)(vbody)

        _stream_chunks((spx_hbm, lab0, lab1, lab2), g0, scratches, per_chunk)
        pltpu.sync_copy(acc, part_out.at[wid])

    return step


_step_init = _make_step("init")
_step_upd = _make_step("update")


@functools.partial(
    pl.kernel,
    out_type=(
        jax.ShapeDtypeStruct((B, 5, K), jnp.float32),
        jax.ShapeDtypeStruct((B, 9, HW), jnp.float32),
        jax.ShapeDtypeStruct((B, 1, HW), jnp.int32),
    ),
    mesh=_mesh(),
    compiler_params=_CPARAMS,
    scratch_types=[
        pltpu.VMEM((WPB, 6, K), jnp.float32),
        pltpu.VMEM((6, K), jnp.float32),
        pltpu.VMEM((6, K), jnp.float32),
        pltpu.VMEM((5, K), jnp.float32),
        pltpu.VMEM((9, CH), jnp.float32),
        pltpu.VMEM((CH,), jnp.int32),
    ] + _IN_CHUNKS,
)
def _final(part_in, spx_hbm, lab0, lab1, lab2,
           spf_out, assoc_out, fidx_out,
           part8, sums, tab, spf, ab, fb, *scratches):
    wid = _worker_id()
    b = wid // WPB
    g0 = wid * PW
    pix_base0 = (wid % WPB) * PW
    _reduce_partials(part_in, b, part8, sums)
    _build_table(sums, tab, "update")

    @pl.when(wid % WPB == 0)
    def _():
        def sbody(i, _):
            sl = pl.ds(i * 16, 16)
            for c in range(5):
                spf[c, sl] = tab[c, sl] * -0.5
            return 0
        lax.fori_loop(0, K // 16, sbody, 0)
        pltpu.sync_copy(spf, spf_out.at[b])

    def per_chunk(cki, off, bufs):
        def vbody(i):
            s = bufs[0][pl.ds(i * 16, 16)]
            feats = _pixel_feats(pix_base0 + off, i, bufs)
            nidx = _nbr_indices(s)
            dists = _gather_dists(tab, feats, nidx)
            ws = _softmax9(dists)
            sl = pl.ds(i * 16, 16)
            for j in range(9):
                ab[j, sl] = ws[j]
            bestd = dists[0]
            bestn = nidx[0]
            for j in range(1, 9):
                lt = dists[j] < bestd
                bestd = jnp.where(lt, dists[j], bestd)
                bestn = jnp.where(lt, nidx[j], bestn)
            fb[sl] = bestn
        plsc.parallel_loop(0, NV, unroll=2)(vbody)
        for j in range(9):
            pltpu.sync_copy(ab.at[j], assoc_out.at[b, j, pl.ds(pix_base0 + off, CH)])
        pltpu.sync_copy(fb, fidx_out.at[b, 0, pl.ds(pix_base0 + off, CH)])

    _stream_chunks((spx_hbm, lab0, lab1, lab2), g0, scratches, per_chunk)


def kernel(img_lab, init_spIndx):
    init_spIndx = init_spIndx.astype(jnp.int32)
    spx = init_spIndx.reshape(B * HW)
    lab = (img_lab * LAB_SCALE).reshape(B, 3, HW)
    lab0 = lab[:, 0].reshape(B * HW)
    lab1 = lab[:, 1].reshape(B * HW)
    lab2 = lab[:, 2].reshape(B * HW)

    p = _init_gather(spx, lab0, lab1, lab2)
    p = _step_init(p, spx, lab0, lab1, lab2)
    for _ in range(3):
        p = _step_upd(p, spx, lab0, lab1, lab2)
    spf, assoc, fidx = _final(p, spx, lab0, lab1, lab2)

    yv = jnp.arange(H, dtype=jnp.float32) * YX_SCALE
    xv = jnp.arange(W, dtype=jnp.float32) * YX_SCALE
    Y = jnp.broadcast_to(yv.reshape(1, 1, H, 1), (B, 1, H, W))
    X = jnp.broadcast_to(xv.reshape(1, 1, 1, W), (B, 1, H, W))
    pFeat = jnp.concatenate([Y, X, LAB_SCALE * img_lab], axis=1)
    return (pFeat, spf, assoc.reshape(B, 9, H, W), fidx.reshape(B, 1, H, W))


# speedup vs baseline: 1.2060x; 1.2060x over previous
import functools

import jax
import jax.numpy as jnp
from jax import lax
from jax.experimental import pallas as pl
from jax.experimental.pallas import tpu as pltpu
from jax.experimental.pallas import tpu_sc as plsc

B, H, W = 4, 512, 512
HW = H * W
KH = KW = 32
K = KH * KW
YX_SCALE = KH / (0.4 * H)
LAB_SCALE = 0.26

NC, NS = 2, 16
NW = NC * NS
PW = (B * HW) // NW
WPB = NW // B
CH = 4096
NCHUNK = PW // CH
NV = CH // 16

_OFFS = ((-1, -1), (-1, 0), (-1, 1), (0, -1), (0, 0), (0, 1), (1, -1), (1, 0), (1, 1))


def _mesh():
    return plsc.VectorSubcoreMesh(core_axis_name="c", subcore_axis_name="s")


_CPARAMS = pltpu.CompilerParams(
    use_tc_tiling_on_sc=False, needs_layout_passes=False)


def _worker_id():
    return lax.axis_index("s") * NC + lax.axis_index("c")


def _tree_sum(xs):
    while len(xs) > 1:
        xs = [xs[i] + xs[i + 1] for i in range(0, len(xs) - 1, 2)] \
            + ([xs[-1]] if len(xs) % 2 else [])
    return xs[0]


def _tree_min(xs):
    while len(xs) > 1:
        xs = [jnp.minimum(xs[i], xs[i + 1]) for i in range(0, len(xs) - 1, 2)] \
            + ([xs[-1]] if len(xs) % 2 else [])
    return xs[0]


def _nbr_indices(s):
    sh = lax.shift_right_logical(s, 5)
    sw = lax.bitwise_and(s, KW - 1)
    out = []
    for dh, dw in _OFFS:
        nh = sh
        if dh < 0:
            nh = jnp.maximum(sh - 1, 0)
        elif dh > 0:
            nh = jnp.minimum(sh + 1, KH - 1)
        nw = sw
        if dw < 0:
            nw = jnp.maximum(sw - 1, 0)
        elif dw > 0:
            nw = jnp.minimum(sw + 1, KW - 1)
        out.append(lax.bitwise_or(lax.shift_left(nh, 5), nw))
    return out


def _pixel_feats(pix_base, i, bufs):
    _, l0b, l1b, l2b = bufs
    lane = lax.iota(jnp.int32, 16)
    pix = (pix_base + i * 16) + lane
    y = lax.shift_right_logical(pix, 9).astype(jnp.float32) * YX_SCALE
    x = lax.bitwise_and(pix, W - 1).astype(jnp.float32) * YX_SCALE
    sl = pl.ds(i * 16, 16)
    return y, x, l0b[sl], l1b[sl], l2b[sl]


def _zero_acc(acc):
    zero = jnp.zeros((16,), jnp.float32)
    for c in range(6):
        def zbody(i, _, c=c):
            acc[c, pl.ds(i * 16, 16)] = zero
            return 0
        lax.fori_loop(0, K // 16, zbody, 0)


def _reduce_partials(part_in, b, part8, sums):
    pltpu.sync_copy(part_in.at[pl.ds(b * WPB, WPB)], part8)
    for c in range(6):
        def rbody(i, _, c=c):
            sl = pl.ds(i * 16, 16)
            sums[c, sl] = _tree_sum([part8[t, c, sl] for t in range(WPB)])
            return 0
        lax.fori_loop(0, K // 16, rbody, 0)


def _build_table(sums, tab, mode):
    def tbody(i, _):
        sl = pl.ds(i * 16, 16)
        den = sums[5, sl]
        if mode == "init":
            den = jnp.maximum(den, 1e-12)
        else:
            den = den + 1e-10
        r = 1.0 / den
        h = None
        for c in range(5):
            t = sums[c, sl] * r
            tab[c, sl] = -2.0 * t
            t2 = t * t
            h = t2 if h is None else h + t2
        tab[5, sl] = h
        return 0
    lax.fori_loop(0, K // 16, tbody, 0)


def _gather_dists(tab, feats, nidx):
    dists = []
    for n in nidx:
        terms = [plsc.load_gather(tab, [jnp.full((16,), 5, jnp.int32), n])]
        for c, f in enumerate(feats):
            g = plsc.load_gather(tab, [jnp.full((16,), c, jnp.int32), n])
            terms.append(f * g)
        dists.append(_tree_sum(terms))
    return dists


def _softmax9(dists):
    m = _tree_min(dists)
    es = [jnp.exp(m - d) for d in dists]
    r = 1.0 / _tree_sum(es)
    return [e * r for e in es]


_IN_CHUNKS = [
    pltpu.VMEM((CH,), jnp.int32),
    pltpu.VMEM((CH,), jnp.float32),
    pltpu.VMEM((CH,), jnp.float32),
    pltpu.VMEM((CH,), jnp.float32),
]


def _stream_chunks(srcs, g0, scratches, per_chunk):
    bufs = tuple(scratches[0:4])

    def chunk_body(cki, _):
        off = cki * CH
        for src, dst in zip(srcs, bufs):
            pltpu.sync_copy(src.at[pl.ds(g0 + off, CH)], dst)
        per_chunk(cki, off, bufs)
        return 0

    lax.fori_loop(0, NCHUNK, chunk_body, 0)


@functools.partial(
    pl.kernel,
    out_type=jax.ShapeDtypeStruct((NW, 6, K), jnp.float32),
    mesh=_mesh(),
    compiler_params=_CPARAMS,
    scratch_types=[pltpu.VMEM((6, K), jnp.float32)] + _IN_CHUNKS,
)
def _init_gather(spx_hbm, lab0, lab1, lab2, part_out, acc, *scratches):
    wid = _worker_id()
    g0 = wid * PW
    pix_base0 = (wid % WPB) * PW
    _zero_acc(acc)
    one = jnp.ones((16,), jnp.float32)
    c5 = jnp.full((16,), 5, jnp.int32)

    def per_chunk(cki, off, bufs):
        def vbody(i, _):
            s = bufs[0][pl.ds(i * 16, 16)]
            feats = _pixel_feats(pix_base0 + off, i, bufs)
            for c, f in enumerate(feats):
                plsc.addupdate_scatter(acc, [jnp.full((16,), c, jnp.int32), s], f)
            plsc.addupdate_scatter(acc, [c5, s], one)
            return 0
        lax.fori_loop(0, NV, vbody, 0)

    _stream_chunks((spx_hbm, lab0, lab1, lab2), g0, scratches, per_chunk)
    pltpu.sync_copy(acc, part_out.at[wid])


def _make_step(mode):
    @functools.partial(
        pl.kernel,
        out_type=jax.ShapeDtypeStruct((NW, 6, K), jnp.float32),
        mesh=_mesh(),
        compiler_params=_CPARAMS,
        scratch_types=[
            pltpu.VMEM((WPB, 6, K), jnp.float32),
            pltpu.VMEM((6, K), jnp.float32),
            pltpu.VMEM((6, K), jnp.float32),
            pltpu.VMEM((6, K), jnp.float32),
        ] + _IN_CHUNKS,
    )
    def step(part_in, spx_hbm, lab0, lab1, lab2, part_out,
             part8, sums, tab, acc, *scratches):
        wid = _worker_id()
        b = wid // WPB
        g0 = wid * PW
        pix_base0 = (wid % WPB) * PW
        _reduce_partials(part_in, b, part8, sums)
        _build_table(sums, tab, mode)
        _zero_acc(acc)
        c5 = jnp.full((16,), 5, jnp.int32)

        def per_chunk(cki, off, bufs):
            def vbody(i):
                s = bufs[0][pl.ds(i * 16, 16)]
                feats = _pixel_feats(pix_base0 + off, i, bufs)
                nidx = _nbr_indices(s)
                dists = _gather_dists(tab, feats, nidx)
                ws = _softmax9(dists)
                for j, n in enumerate(nidx):
                    w = ws[j]
                    plsc.addupdate_scatter(acc, [c5, n], w)
                    for c, f in enumerate(feats):
                        plsc.addupdate_scatter(
                            acc, [jnp.full((16,), c, jnp.int32), n], w * f)
            plsc.parallel_loop(0, NV, unroll=1)(vbody)

        _stream_chunks((spx_hbm, lab0, lab1, lab2), g0, scratches, per_chunk)
        pltpu.sync_copy(acc, part_out.at[wid])

    return step


_step_init = _make_step("init")
_step_upd = _make_step("update")


@functools.partial(
    pl.kernel,
    out_type=(
        jax.ShapeDtypeStruct((B, 5, K), jnp.float32),
        jax.ShapeDtypeStruct((B, 9, HW), jnp.float32),
        jax.ShapeDtypeStruct((B, 1, HW), jnp.int32),
    ),
    mesh=_mesh(),
    compiler_params=_CPARAMS,
    scratch_types=[
        pltpu.VMEM((WPB, 6, K), jnp.float32),
        pltpu.VMEM((6, K), jnp.float32),
        pltpu.VMEM((6, K), jnp.float32),
        pltpu.VMEM((5, K), jnp.float32),
        pltpu.VMEM((9, CH), jnp.float32),
        pltpu.VMEM((CH,), jnp.int32),
    ] + _IN_CHUNKS,
)
def _final(part_in, spx_hbm, lab0, lab1, lab2,
           spf_out, assoc_out, fidx_out,
           part8, sums, tab, spf, ab, fb, *scratches):
    wid = _worker_id()
    b = wid // WPB
    g0 = wid * PW
    pix_base0 = (wid % WPB) * PW
    _reduce_partials(part_in, b, part8, sums)
    _build_table(sums, tab, "update")

    @pl.when(wid % WPB == 0)
    def _():
        def sbody(i, _):
            sl = pl.ds(i * 16, 16)
            for c in range(5):
                spf[c, sl] = tab[c, sl] * -0.5
            return 0
        lax.fori_loop(0, K // 16, sbody, 0)
        pltpu.sync_copy(spf, spf_out.at[b])

    def per_chunk(cki, off, bufs):
        def vbody(i):
            s = bufs[0][pl.ds(i * 16, 16)]
            feats = _pixel_feats(pix_base0 + off, i, bufs)
            nidx = _nbr_indices(s)
            dists = _gather_dists(tab, feats, nidx)
            ws = _softmax9(dists)
            sl = pl.ds(i * 16, 16)
            for j in range(9):
                ab[j, sl] = ws[j]
            bestd = dists[0]
            bestn = nidx[0]
            for j in range(1, 9):
                lt = dists[j] < bestd
                bestd = jnp.where(lt, dists[j], bestd)
                bestn = jnp.where(lt, nidx[j], bestn)
            fb[sl] = bestn
        plsc.parallel_loop(0, NV, unroll=1)(vbody)
        for j in range(9):
            pltpu.sync_copy(ab.at[j], assoc_out.at[b, j, pl.ds(pix_base0 + off, CH)])
        pltpu.sync_copy(fb, fidx_out.at[b, 0, pl.ds(pix_base0 + off, CH)])

    _stream_chunks((spx_hbm, lab0, lab1, lab2), g0, scratches, per_chunk)


def kernel(img_lab, init_spIndx):
    init_spIndx = init_spIndx.astype(jnp.int32)
    spx = init_spIndx.reshape(B * HW)
    lab = (img_lab * LAB_SCALE).reshape(B, 3, HW)
    lab0 = lab[:, 0].reshape(B * HW)
    lab1 = lab[:, 1].reshape(B * HW)
    lab2 = lab[:, 2].reshape(B * HW)

    p = _init_gather(spx, lab0, lab1, lab2)
    p = _step_init(p, spx, lab0, lab1, lab2)
    for _ in range(3):
        p = _step_upd(p, spx, lab0, lab1, lab2)
    spf, assoc, fidx = _final(p, spx, lab0, lab1, lab2)

    yv = jnp.arange(H, dtype=jnp.float32) * YX_SCALE
    xv = jnp.arange(W, dtype=jnp.float32) * YX_SCALE
    Y = jnp.broadcast_to(yv.reshape(1, 1, H, 1), (B, 1, H, W))
    X = jnp.broadcast_to(xv.reshape(1, 1, 1, W), (B, 1, H, W))
    pFeat = jnp.concatenate([Y, X, LAB_SCALE * img_lab], axis=1)
    return (pFeat, spf, assoc.reshape(B, 9, H, W), fidx.reshape(B, 1, H, W))


# speedup vs baseline: 1.7154x; 1.4224x over previous
import functools

import jax
import jax.numpy as jnp
from jax import lax
from jax.experimental import pallas as pl
from jax.experimental.pallas import tpu as pltpu
from jax.experimental.pallas import tpu_sc as plsc

B, H, W = 4, 512, 512
HW = H * W
KH = KW = 32
K = KH * KW
YX_SCALE = KH / (0.4 * H)
LAB_SCALE = 0.26

NC, NS = 2, 16
NW = NC * NS
PW = (B * HW) // NW
WPB = NW // B
CH = 4096
CHB = 8192

_OFFS = ((-1, -1), (-1, 0), (-1, 1), (0, -1), (0, 0), (0, 1), (1, -1), (1, 0), (1, 1))


def _mesh():
    return plsc.VectorSubcoreMesh(core_axis_name="c", subcore_axis_name="s")


_CPARAMS = pltpu.CompilerParams(
    use_tc_tiling_on_sc=False, needs_layout_passes=False)


def _worker_id():
    return lax.axis_index("s") * NC + lax.axis_index("c")


def _tree_sum(xs):
    while len(xs) > 1:
        xs = [xs[i] + xs[i + 1] for i in range(0, len(xs) - 1, 2)] \
            + ([xs[-1]] if len(xs) % 2 else [])
    return xs[0]


def _tree_min(xs):
    while len(xs) > 1:
        xs = [jnp.minimum(xs[i], xs[i + 1]) for i in range(0, len(xs) - 1, 2)] \
            + ([xs[-1]] if len(xs) % 2 else [])
    return xs[0]


def _nbr_indices(s):
    sh = lax.shift_right_logical(s, 5)
    sw = lax.bitwise_and(s, KW - 1)
    out = []
    for dh, dw in _OFFS:
        nh = sh
        if dh < 0:
            nh = jnp.maximum(sh - 1, 0)
        elif dh > 0:
            nh = jnp.minimum(sh + 1, KH - 1)
        nw = sw
        if dw < 0:
            nw = jnp.maximum(sw - 1, 0)
        elif dw > 0:
            nw = jnp.minimum(sw + 1, KW - 1)
        out.append(lax.bitwise_or(lax.shift_left(nh, 5), nw))
    return out


def _pixel_feats(pix_base, i, bufs):
    _, l0b, l1b, l2b = bufs
    lane = lax.iota(jnp.int32, 16)
    pix = (pix_base + i * 16) + lane
    y = lax.shift_right_logical(pix, 9).astype(jnp.float32) * YX_SCALE
    x = lax.bitwise_and(pix, W - 1).astype(jnp.float32) * YX_SCALE
    sl = pl.ds(i * 16, 16)
    return y, x, l0b[sl], l1b[sl], l2b[sl]


def _zero_acc(acc):
    zero = jnp.zeros((16,), jnp.float32)
    for c in range(6):
        def zbody(i, _, c=c):
            acc[c, pl.ds(i * 16, 16)] = zero
            return 0
        lax.fori_loop(0, K // 16, zbody, 0)


def _reduce_partials(part_in, b, part8, sums):
    pltpu.sync_copy(part_in.at[pl.ds(b * WPB, WPB)], part8)
    for c in range(6):
        def rbody(i, _, c=c):
            sl = pl.ds(i * 16, 16)
            sums[c, sl] = _tree_sum([part8[t, c, sl] for t in range(WPB)])
            return 0
        lax.fori_loop(0, K // 16, rbody, 0)


def _build_table(sums, tab, mode):
    def tbody(i, _):
        sl = pl.ds(i * 16, 16)
        den = sums[5, sl]
        if mode == "init":
            den = jnp.maximum(den, 1e-12)
        else:
            den = den + 1e-10
        r = 1.0 / den
        h = None
        for c in range(5):
            t = sums[c, sl] * r
            tab[c, sl] = -2.0 * t
            t2 = t * t
            h = t2 if h is None else h + t2
        tab[5, sl] = h
        return 0
    lax.fori_loop(0, K // 16, tbody, 0)


def _gather_dists(tab, feats, nidx):
    dists = []
    for n in nidx:
        terms = [plsc.load_gather(tab, [jnp.full((16,), 5, jnp.int32), n])]
        for c, f in enumerate(feats):
            g = plsc.load_gather(tab, [jnp.full((16,), c, jnp.int32), n])
            terms.append(f * g)
        dists.append(_tree_sum(terms))
    return dists


def _softmax9(dists):
    m = _tree_min(dists)
    es = [jnp.exp(m - d) for d in dists]
    r = 1.0 / _tree_sum(es)
    return [e * r for e in es]


def _in_chunks(ch):
    return [
        pltpu.VMEM((ch,), jnp.int32),
        pltpu.VMEM((ch,), jnp.float32),
        pltpu.VMEM((ch,), jnp.float32),
        pltpu.VMEM((ch,), jnp.float32),
    ]


def _stream_chunks(srcs, g0, scratches, per_chunk, ch):
    bufs = tuple(scratches[0:4])

    def chunk_body(cki, _):
        off = cki * ch
        for src, dst in zip(srcs, bufs):
            pltpu.sync_copy(src.at[pl.ds(g0 + off, ch)], dst)
        per_chunk(cki, off, bufs)
        return 0

    lax.fori_loop(0, PW // ch, chunk_body, 0)


@functools.partial(
    pl.kernel,
    out_type=jax.ShapeDtypeStruct((NW, 6, K), jnp.float32),
    mesh=_mesh(),
    compiler_params=_CPARAMS,
    scratch_types=[pltpu.VMEM((6, K), jnp.float32)] + _in_chunks(CHB),
)
def _init_gather(spx_hbm, lab0, lab1, lab2, part_out, acc, *scratches):
    wid = _worker_id()
    g0 = wid * PW
    pix_base0 = (wid % WPB) * PW
    _zero_acc(acc)
    one = jnp.ones((16,), jnp.float32)
    c5 = jnp.full((16,), 5, jnp.int32)

    def per_chunk(cki, off, bufs):
        def vbody(i, _):
            s = bufs[0][pl.ds(i * 16, 16)]
            feats = _pixel_feats(pix_base0 + off, i, bufs)
            for c, f in enumerate(feats):
                plsc.addupdate_scatter(acc, [jnp.full((16,), c, jnp.int32), s], f)
            plsc.addupdate_scatter(acc, [c5, s], one)
            return 0
        lax.fori_loop(0, CHB // 16, vbody, 0)

    _stream_chunks((spx_hbm, lab0, lab1, lab2), g0, scratches, per_chunk, CHB)
    pltpu.sync_copy(acc, part_out.at[wid])


def _make_step(mode):
    @functools.partial(
        pl.kernel,
        out_type=jax.ShapeDtypeStruct((NW, 6, K), jnp.float32),
        mesh=_mesh(),
        compiler_params=_CPARAMS,
        scratch_types=[
            pltpu.VMEM((WPB, 6, K), jnp.float32),
            pltpu.VMEM((6, K), jnp.float32),
            pltpu.VMEM((6, K), jnp.float32),
            pltpu.VMEM((6, K), jnp.float32),
        ] + _in_chunks(CHB),
    )
    def step(part_in, spx_hbm, lab0, lab1, lab2, part_out,
             part8, sums, tab, acc, *scratches):
        wid = _worker_id()
        b = wid // WPB
        g0 = wid * PW
        pix_base0 = (wid % WPB) * PW
        _reduce_partials(part_in, b, part8, sums)
        _build_table(sums, tab, mode)
        _zero_acc(acc)
        c5 = jnp.full((16,), 5, jnp.int32)

        def per_chunk(cki, off, bufs):
            def vbody(i, _):
                s = bufs[0][pl.ds(i * 16, 16)]
                feats = _pixel_feats(pix_base0 + off, i, bufs)
                nidx = _nbr_indices(s)
                dists = _gather_dists(tab, feats, nidx)
                ws = _softmax9(dists)
                for j, n in enumerate(nidx):
                    w = ws[j]
                    plsc.addupdate_scatter(acc, [c5, n], w)
                    for c, f in enumerate(feats):
                        plsc.addupdate_scatter(
                            acc, [jnp.full((16,), c, jnp.int32), n], w * f)
                return 0
            lax.fori_loop(0, CHB // 16, vbody, 0)

        _stream_chunks((spx_hbm, lab0, lab1, lab2), g0, scratches, per_chunk, CHB)
        pltpu.sync_copy(acc, part_out.at[wid])

    return step


_step_init = _make_step("init")
_step_upd = _make_step("update")


@functools.partial(
    pl.kernel,
    out_type=(
        jax.ShapeDtypeStruct((B, 5, K), jnp.float32),
        jax.ShapeDtypeStruct((B, 9, HW), jnp.float32),
        jax.ShapeDtypeStruct((B, 1, HW), jnp.int32),
    ),
    mesh=_mesh(),
    compiler_params=_CPARAMS,
    scratch_types=[
        pltpu.VMEM((WPB, 6, K), jnp.float32),
        pltpu.VMEM((6, K), jnp.float32),
        pltpu.VMEM((6, K), jnp.float32),
        pltpu.VMEM((5, K), jnp.float32),
        pltpu.VMEM((9, CH), jnp.float32),
        pltpu.VMEM((CH,), jnp.int32),
    ] + _in_chunks(CH),
)
def _final(part_in, spx_hbm, lab0, lab1, lab2,
           spf_out, assoc_out, fidx_out,
           part8, sums, tab, spf, ab, fb, *scratches):
    wid = _worker_id()
    b = wid // WPB
    g0 = wid * PW
    pix_base0 = (wid % WPB) * PW
    _reduce_partials(part_in, b, part8, sums)
    _build_table(sums, tab, "update")

    @pl.when(wid % WPB == 0)
    def _():
        def sbody(i, _):
            sl = pl.ds(i * 16, 16)
            for c in range(5):
                spf[c, sl] = tab[c, sl] * -0.5
            return 0
        lax.fori_loop(0, K // 16, sbody, 0)
        pltpu.sync_copy(spf, spf_out.at[b])

    def per_chunk(cki, off, bufs):
        def vbody(i, _):
            s = bufs[0][pl.ds(i * 16, 16)]
            feats = _pixel_feats(pix_base0 + off, i, bufs)
            nidx = _nbr_indices(s)
            dists = _gather_dists(tab, feats, nidx)
            ws = _softmax9(dists)
            sl = pl.ds(i * 16, 16)
            for j in range(9):
                ab[j, sl] = ws[j]
            bestd = dists[0]
            bestn = nidx[0]
            for j in range(1, 9):
                lt = dists[j] < bestd
                bestd = jnp.where(lt, dists[j], bestd)
                bestn = jnp.where(lt, nidx[j], bestn)
            fb[sl] = bestn
            return 0
        lax.fori_loop(0, CH // 16, vbody, 0)
        for j in range(9):
            pltpu.sync_copy(ab.at[j], assoc_out.at[b, j, pl.ds(pix_base0 + off, CH)])
        pltpu.sync_copy(fb, fidx_out.at[b, 0, pl.ds(pix_base0 + off, CH)])

    _stream_chunks((spx_hbm, lab0, lab1, lab2), g0, scratches, per_chunk, CH)


def kernel(img_lab, init_spIndx):
    init_spIndx = init_spIndx.astype(jnp.int32)
    spx = init_spIndx.reshape(B * HW)
    lab = (img_lab * LAB_SCALE).reshape(B, 3, HW)
    lab0 = lab[:, 0].reshape(B * HW)
    lab1 = lab[:, 1].reshape(B * HW)
    lab2 = lab[:, 2].reshape(B * HW)

    p = _init_gather(spx, lab0, lab1, lab2)
    p = _step_init(p, spx, lab0, lab1, lab2)
    for _ in range(3):
        p = _step_upd(p, spx, lab0, lab1, lab2)
    spf, assoc, fidx = _final(p, spx, lab0, lab1, lab2)

    yv = jnp.arange(H, dtype=jnp.float32) * YX_SCALE
    xv = jnp.arange(W, dtype=jnp.float32) * YX_SCALE
    Y = jnp.broadcast_to(yv.reshape(1, 1, H, 1), (B, 1, H, W))
    X = jnp.broadcast_to(xv.reshape(1, 1, 1, W), (B, 1, H, W))
    pFeat = jnp.concatenate([Y, X, LAB_SCALE * img_lab], axis=1)
    return (pFeat, spf, assoc.reshape(B, 9, H, W), fidx.reshape(B, 1, H, W))


# speedup vs baseline: 1.7638x; 1.0282x over previous
import functools

import jax
import jax.numpy as jnp
from jax import lax
from jax.experimental import pallas as pl
from jax.experimental.pallas import tpu as pltpu
from jax.experimental.pallas import tpu_sc as plsc

B, H, W = 4, 512, 512
HW = H * W
KH = KW = 32
K = KH * KW
YX_SCALE = KH / (0.4 * H)
LAB_SCALE = 0.26

NC, NS = 2, 16
NW = NC * NS
PW = (B * HW) // NW
WPB = NW // B
CH = 4096
CHB = 8192

_OFFS = ((-1, -1), (-1, 0), (-1, 1), (0, -1), (0, 0), (0, 1), (1, -1), (1, 0), (1, 1))


def _mesh():
    return plsc.VectorSubcoreMesh(core_axis_name="c", subcore_axis_name="s")


_CPARAMS = pltpu.CompilerParams(
    use_tc_tiling_on_sc=False, needs_layout_passes=False)


def _worker_id():
    return lax.axis_index("s") * NC + lax.axis_index("c")


def _tree_sum(xs):
    while len(xs) > 1:
        xs = [xs[i] + xs[i + 1] for i in range(0, len(xs) - 1, 2)] \
            + ([xs[-1]] if len(xs) % 2 else [])
    return xs[0]


def _tree_min(xs):
    while len(xs) > 1:
        xs = [jnp.minimum(xs[i], xs[i + 1]) for i in range(0, len(xs) - 1, 2)] \
            + ([xs[-1]] if len(xs) % 2 else [])
    return xs[0]


def _nbr_indices(s):
    sh = lax.shift_right_logical(s, 5)
    sw = lax.bitwise_and(s, KW - 1)
    out = []
    for dh, dw in _OFFS:
        nh = sh
        if dh < 0:
            nh = jnp.maximum(sh - 1, 0)
        elif dh > 0:
            nh = jnp.minimum(sh + 1, KH - 1)
        nw = sw
        if dw < 0:
            nw = jnp.maximum(sw - 1, 0)
        elif dw > 0:
            nw = jnp.minimum(sw + 1, KW - 1)
        out.append(lax.bitwise_or(lax.shift_left(nh, 5), nw))
    return out


def _pixel_feats(pix_base, i, bufs):
    _, l0b, l1b, l2b = bufs
    lane = lax.iota(jnp.int32, 16)
    pix = (pix_base + i * 16) + lane
    y = lax.shift_right_logical(pix, 9).astype(jnp.float32) * YX_SCALE
    x = lax.bitwise_and(pix, W - 1).astype(jnp.float32) * YX_SCALE
    sl = pl.ds(i * 16, 16)
    return y, x, l0b[sl], l1b[sl], l2b[sl]


def _zero_acc(acc):
    zero = jnp.zeros((16,), jnp.float32)
    for c in range(6):
        def zbody(i, _, c=c):
            acc[c, pl.ds(i * 16, 16)] = zero
            return 0
        lax.fori_loop(0, K // 16, zbody, 0)


def _reduce_partials(part_in, b, part8, sums):
    pltpu.sync_copy(part_in.at[pl.ds(b * WPB, WPB)], part8)
    for c in range(6):
        def rbody(i, _, c=c):
            sl = pl.ds(i * 16, 16)
            sums[c, sl] = _tree_sum([part8[t, c, sl] for t in range(WPB)])
            return 0
        lax.fori_loop(0, K // 16, rbody, 0)


def _build_table(sums, tab, mode):
    def tbody(i, _):
        sl = pl.ds(i * 16, 16)
        den = sums[5, sl]
        if mode == "init":
            den = jnp.maximum(den, 1e-12)
        else:
            den = den + 1e-10
        r = 1.0 / den
        h = None
        for c in range(5):
            t = sums[c, sl] * r
            tab[c, sl] = -2.0 * t
            t2 = t * t
            h = t2 if h is None else h + t2
        tab[5, sl] = h
        return 0
    lax.fori_loop(0, K // 16, tbody, 0)


def _gather_dists(tab, feats, nidx):
    dists = []
    for n in nidx:
        terms = [plsc.load_gather(tab, [jnp.full((16,), 5, jnp.int32), n])]
        for c, f in enumerate(feats):
            g = plsc.load_gather(tab, [jnp.full((16,), c, jnp.int32), n])
            terms.append(f * g)
        dists.append(_tree_sum(terms))
    return dists


def _softmax9(dists):
    m = _tree_min(dists)
    es = [jnp.exp(m - d) for d in dists]
    r = 1.0 / _tree_sum(es)
    return [e * r for e in es]


def _in_chunks(ch):
    return [
        pltpu.VMEM((ch,), jnp.int32),
        pltpu.VMEM((ch,), jnp.float32),
        pltpu.VMEM((ch,), jnp.float32),
        pltpu.VMEM((ch,), jnp.float32),
        pltpu.SemaphoreType.DMA,
    ]


def _stream_chunks(srcs, g0, scratches, per_chunk, ch):
    bufs = tuple(scratches[0:4])
    sem = scratches[4]

    def chunk_body(cki, _):
        off = cki * ch
        copies = [pltpu.async_copy(src.at[pl.ds(g0 + off, ch)], dst, sem)
                  for src, dst in zip(srcs, bufs)]
        for cp in copies:
            cp.wait()
        per_chunk(cki, off, bufs)
        return 0

    lax.fori_loop(0, PW // ch, chunk_body, 0)


@functools.partial(
    pl.kernel,
    out_type=jax.ShapeDtypeStruct((NW, 6, K), jnp.float32),
    mesh=_mesh(),
    compiler_params=_CPARAMS,
    scratch_types=[pltpu.VMEM((6, K), jnp.float32)] + _in_chunks(CHB),
)
def _init_gather(spx_hbm, lab0, lab1, lab2, part_out, acc, *scratches):
    wid = _worker_id()
    g0 = wid * PW
    pix_base0 = (wid % WPB) * PW
    _zero_acc(acc)
    one = jnp.ones((16,), jnp.float32)
    c5 = jnp.full((16,), 5, jnp.int32)

    def per_chunk(cki, off, bufs):
        def vbody(i, _):
            s = bufs[0][pl.ds(i * 16, 16)]
            feats = _pixel_feats(pix_base0 + off, i, bufs)
            for c, f in enumerate(feats):
                plsc.addupdate_scatter(acc, [jnp.full((16,), c, jnp.int32), s], f)
            plsc.addupdate_scatter(acc, [c5, s], one)
            return 0
        lax.fori_loop(0, CHB // 16, vbody, 0)

    _stream_chunks((spx_hbm, lab0, lab1, lab2), g0, scratches, per_chunk, CHB)
    pltpu.sync_copy(acc, part_out.at[wid])


def _make_step(mode):
    @functools.partial(
        pl.kernel,
        out_type=jax.ShapeDtypeStruct((NW, 6, K), jnp.float32),
        mesh=_mesh(),
        compiler_params=_CPARAMS,
        scratch_types=[
            pltpu.VMEM((WPB, 6, K), jnp.float32),
            pltpu.VMEM((6, K), jnp.float32),
            pltpu.VMEM((6, K), jnp.float32),
            pltpu.VMEM((6, K), jnp.float32),
        ] + _in_chunks(CHB),
    )
    def step(part_in, spx_hbm, lab0, lab1, lab2, part_out,
             part8, sums, tab, acc, *scratches):
        wid = _worker_id()
        b = wid // WPB
        g0 = wid * PW
        pix_base0 = (wid % WPB) * PW
        _reduce_partials(part_in, b, part8, sums)
        _build_table(sums, tab, mode)
        _zero_acc(acc)
        c5 = jnp.full((16,), 5, jnp.int32)

        def per_chunk(cki, off, bufs):
            def vbody(i, _):
                s = bufs[0][pl.ds(i * 16, 16)]
                feats = _pixel_feats(pix_base0 + off, i, bufs)
                nidx = _nbr_indices(s)
                dists = _gather_dists(tab, feats, nidx)
                ws = _softmax9(dists)
                for j, n in enumerate(nidx):
                    w = ws[j]
                    plsc.addupdate_scatter(acc, [c5, n], w)
                    for c, f in enumerate(feats):
                        plsc.addupdate_scatter(
                            acc, [jnp.full((16,), c, jnp.int32), n], w * f)
                return 0
            lax.fori_loop(0, CHB // 16, vbody, 0)

        _stream_chunks((spx_hbm, lab0, lab1, lab2), g0, scratches, per_chunk, CHB)
        pltpu.sync_copy(acc, part_out.at[wid])

    return step


_step_init = _make_step("init")
_step_upd = _make_step("update")


@functools.partial(
    pl.kernel,
    out_type=(
        jax.ShapeDtypeStruct((B, 5, K), jnp.float32),
        jax.ShapeDtypeStruct((B, 9, HW), jnp.float32),
        jax.ShapeDtypeStruct((B, 1, HW), jnp.int32),
    ),
    mesh=_mesh(),
    compiler_params=_CPARAMS,
    scratch_types=[
        pltpu.VMEM((WPB, 6, K), jnp.float32),
        pltpu.VMEM((6, K), jnp.float32),
        pltpu.VMEM((6, K), jnp.float32),
        pltpu.VMEM((5, K), jnp.float32),
        pltpu.VMEM((9, CH), jnp.float32),
        pltpu.VMEM((CH,), jnp.int32),
    ] + _in_chunks(CH),
)
def _final(part_in, spx_hbm, lab0, lab1, lab2,
           spf_out, assoc_out, fidx_out,
           part8, sums, tab, spf, ab, fb, *scratches):
    wid = _worker_id()
    b = wid // WPB
    g0 = wid * PW
    pix_base0 = (wid % WPB) * PW
    _reduce_partials(part_in, b, part8, sums)
    _build_table(sums, tab, "update")

    @pl.when(wid % WPB == 0)
    def _():
        def sbody(i, _):
            sl = pl.ds(i * 16, 16)
            for c in range(5):
                spf[c, sl] = tab[c, sl] * -0.5
            return 0
        lax.fori_loop(0, K // 16, sbody, 0)
        pltpu.sync_copy(spf, spf_out.at[b])

    def per_chunk(cki, off, bufs):
        def vbody(i, _):
            s = bufs[0][pl.ds(i * 16, 16)]
            feats = _pixel_feats(pix_base0 + off, i, bufs)
            nidx = _nbr_indices(s)
            dists = _gather_dists(tab, feats, nidx)
            ws = _softmax9(dists)
            sl = pl.ds(i * 16, 16)
            for j in range(9):
                ab[j, sl] = ws[j]
            bestd = dists[0]
            bestn = nidx[0]
            for j in range(1, 9):
                lt = dists[j] < bestd
                bestd = jnp.where(lt, dists[j], bestd)
                bestn = jnp.where(lt, nidx[j], bestn)
            fb[sl] = bestn
            return 0
        lax.fori_loop(0, CH // 16, vbody, 0)
        for j in range(9):
            pltpu.sync_copy(ab.at[j], assoc_out.at[b, j, pl.ds(pix_base0 + off, CH)])
        pltpu.sync_copy(fb, fidx_out.at[b, 0, pl.ds(pix_base0 + off, CH)])

    _stream_chunks((spx_hbm, lab0, lab1, lab2), g0, scratches, per_chunk, CH)


def kernel(img_lab, init_spIndx):
    init_spIndx = init_spIndx.astype(jnp.int32)
    spx = init_spIndx.reshape(B * HW)
    lab = (img_lab * LAB_SCALE).reshape(B, 3, HW)
    lab0 = lab[:, 0].reshape(B * HW)
    lab1 = lab[:, 1].reshape(B * HW)
    lab2 = lab[:, 2].reshape(B * HW)

    p = _init_gather(spx, lab0, lab1, lab2)
    p = _step_init(p, spx, lab0, lab1, lab2)
    for _ in range(3):
        p = _step_upd(p, spx, lab0, lab1, lab2)
    spf, assoc, fidx = _final(p, spx, lab0, lab1, lab2)

    yv = jnp.arange(H, dtype=jnp.float32) * YX_SCALE
    xv = jnp.arange(W, dtype=jnp.float32) * YX_SCALE
    Y = jnp.broadcast_to(yv.reshape(1, 1, H, 1), (B, 1, H, W))
    X = jnp.broadcast_to(xv.reshape(1, 1, 1, W), (B, 1, H, W))
    pFeat = jnp.concatenate([Y, X, LAB_SCALE * img_lab], axis=1)
    return (pFeat, spf, assoc.reshape(B, 9, H, W), fidx.reshape(B, 1, H, W))
